# baseline (speedup 1.0000x reference)
import functools
import math

import jax
import jax.numpy as jnp
from jax import lax
from jax.experimental import pallas as pl
from jax.experimental.pallas import tpu as pltpu
from jax.experimental.pallas import tpu_sc as plsc

_SCALE = 30.0
_MARGIN = 0.5
_COS_M = math.cos(_MARGIN)
_SIN_M = math.sin(_MARGIN)
_TH = math.cos(math.pi - _MARGIN)
_MM = math.sin(math.pi - _MARGIN) * _MARGIN
_LOG2E = 1.4426950408889634
_A2 = _SCALE * _LOG2E

_B = 1024
_C = 100000

_NC = 2
_NS = 16
_NW = _NC * _NS
_BPW = _B // _NW

@functools.cache
def _sc_gather_fn():
    mesh = plsc.VectorSubcoreMesh(core_axis_name="c", subcore_axis_name="s")

    @functools.partial(
        pl.kernel,
        mesh=mesh,
        out_type=jax.ShapeDtypeStruct((_B,), jnp.float32),
        scratch_types=[
            pltpu.VMEM((_BPW,), jnp.int32),
            pltpu.VMEM((_BPW,), jnp.int32),
            pltpu.VMEM((_BPW,), jnp.float32),
            pltpu.SemaphoreType.DMA,
        ],
    )
    def _sc_gather(flat_hbm, labels_hbm, out_hbm, lbl_v, idx_v, val_v, sem):
        wid = lax.axis_index("s") * _NC + lax.axis_index("c")
        base = wid * _BPW
        pltpu.sync_copy(labels_hbm.at[pl.ds(base, _BPW)], lbl_v)
        for j in range(_BPW // 16):
            lbl = lbl_v[pl.ds(j * 16, 16)]
            rows = lax.iota(jnp.int32, 16) + (base + j * 16)
            idx_v[pl.ds(j * 16, 16)] = rows * _C + lbl
        pltpu.async_copy(flat_hbm.at[idx_v], val_v, sem).wait()
        pltpu.sync_copy(val_v, out_hbm.at[pl.ds(base, _BPW)])

    return _sc_gather


_RPT = _B // _NW
_CH = _C // 2
_NCHK = 2 * _RPT
_UNROLL = 5
_L = 16


@functools.cache
def _sc_rowsum_fn():
    mesh = plsc.VectorSubcoreMesh(core_axis_name="c", subcore_axis_name="s")

    @functools.partial(
        pl.kernel,
        mesh=mesh,
        out_type=jax.ShapeDtypeStruct((_B * _L,), jnp.float32),
        scratch_types=[
            pltpu.VMEM((_CH,), jnp.float32),
            pltpu.VMEM((_CH,), jnp.float32),
            pltpu.VMEM((_RPT * _L,), jnp.float32),
            pltpu.SemaphoreType.DMA,
            pltpu.SemaphoreType.DMA,
        ],
    )
    def _sc_rowsum(flat_hbm, out_hbm, buf0, buf1, acc_v, sem0, sem1):
        wid = lax.axis_index("s") * _NC + lax.axis_index("c")
        base = wid * (_RPT * _C)
        bufs = (buf0, buf1)
        sems = (sem0, sem1)

        def _start(j):
            pltpu.make_async_copy(
                flat_hbm.at[pl.ds(base + j * _CH, _CH)], bufs[j % 2], sems[j % 2]
            ).start()

        def _wait(j):
            pltpu.make_async_copy(
                flat_hbm.at[pl.ds(base + j * _CH, _CH)], bufs[j % 2], sems[j % 2]
            ).wait()

        _start(0)
        for r in range(_RPT):
            accs = tuple(jnp.zeros((_L,), jnp.float32) for _ in range(_UNROLL))
            for h in range(2):
                j = 2 * r + h
                if j + 1 < _NCHK:
                    _start(j + 1)
                _wait(j)
                buf = bufs[j % 2]

                def body(i, accs, buf=buf):
                    b = i * (_UNROLL * _L)
                    return tuple(
                        a + jnp.exp(buf[pl.ds(b + k * _L, _L)] * _SCALE - _SCALE)
                        for k, a in enumerate(accs)
                    )

                accs = lax.fori_loop(0, _CH // (_UNROLL * _L), body, accs)
            tot = accs[0]
            for a in accs[1:]:
                tot = tot + a
            acc_v[pl.ds(r * _L, _L)] = tot
        pltpu.sync_copy(acc_v, out_hbm.at[pl.ds(wid * _RPT * _L, _RPT * _L)])

    return _sc_rowsum


def _combine_body(g_ref, s_ref, o_ref):
    g = g_ref[...]
    s0 = jnp.sum(s_ref[...], axis=1, keepdims=True)
    c = jnp.clip(g, -1.0 + 1e-07, 1.0 - 1e-07)
    sin_t = jnp.sqrt(1.0 - c * c)
    ctm = c * _COS_M - sin_t * _SIN_M
    ctm = jnp.where(c > _TH, ctm, c - _MM)
    t = _SCALE * ctm
    s1 = s0 - jnp.exp2(g * _A2 - _A2) + jnp.exp2(t * _LOG2E - _A2)
    nll = _SCALE + jnp.log(s1) - t
    o_ref[...] = jnp.sum(nll, axis=0, keepdims=True) * (1.0 / _B)


def _tc_combine(g, s0):
    return pl.pallas_call(
        _combine_body,
        out_shape=jax.ShapeDtypeStruct((1, 1), jnp.float32),
    )(g, s0)


def kernel(cosine, labels):
    labels = labels.astype(jnp.int32)
    flat = cosine.reshape(-1)
    g = _sc_gather_fn()(flat, labels)
    s0p = _sc_rowsum_fn()(flat)
    out = _tc_combine(g.reshape(_B, 1), s0p.reshape(_B, _L))
    return out[0, 0]


# speedup vs baseline: 2.0975x; 2.0975x over previous
import functools
import math

import jax
import jax.numpy as jnp
from jax import lax
from jax.experimental import pallas as pl
from jax.experimental.pallas import tpu as pltpu
from jax.experimental.pallas import tpu_sc as plsc

_SCALE = 30.0
_MARGIN = 0.5
_COS_M = math.cos(_MARGIN)
_SIN_M = math.sin(_MARGIN)
_TH = math.cos(math.pi - _MARGIN)
_MM = math.sin(math.pi - _MARGIN) * _MARGIN
_LOG2E = 1.4426950408889634
_A2 = _SCALE * _LOG2E

_B = 1024
_C = 100000

_NC = 2
_NS = 16
_NW = _NC * _NS
_BPW = _B // _NW

@functools.cache
def _sc_gather_fn():
    mesh = plsc.VectorSubcoreMesh(core_axis_name="c", subcore_axis_name="s")

    @functools.partial(
        pl.kernel,
        mesh=mesh,
        out_type=jax.ShapeDtypeStruct((_B,), jnp.float32),
        scratch_types=[
            pltpu.VMEM((_BPW,), jnp.int32),
            pltpu.VMEM((_BPW,), jnp.int32),
            pltpu.VMEM((_BPW,), jnp.float32),
            pltpu.SemaphoreType.DMA,
        ],
    )
    def _sc_gather(flat_hbm, labels_hbm, out_hbm, lbl_v, idx_v, val_v, sem):
        wid = lax.axis_index("s") * _NC + lax.axis_index("c")
        base = wid * _BPW
        pltpu.sync_copy(labels_hbm.at[pl.ds(base, _BPW)], lbl_v)
        for j in range(_BPW // 16):
            lbl = lbl_v[pl.ds(j * 16, 16)]
            rows = lax.iota(jnp.int32, 16) + (base + j * 16)
            idx_v[pl.ds(j * 16, 16)] = rows * _C + lbl
        pltpu.async_copy(flat_hbm.at[idx_v], val_v, sem).wait()
        pltpu.sync_copy(val_v, out_hbm.at[pl.ds(base, _BPW)])

    return _sc_gather


_RPT = _B // _NW
_CH = _C // 2
_NCHK = 2 * _RPT
_UNROLL = 5
_L = 16


@functools.cache
def _sc_rowsum_fn():
    mesh = plsc.VectorSubcoreMesh(core_axis_name="c", subcore_axis_name="s")

    @functools.partial(
        pl.kernel,
        mesh=mesh,
        out_type=jax.ShapeDtypeStruct((_B * _L,), jnp.float32),
        scratch_types=[
            pltpu.VMEM((_CH,), jnp.float32),
            pltpu.VMEM((_CH,), jnp.float32),
            pltpu.VMEM((_RPT * _L,), jnp.float32),
            pltpu.SemaphoreType.DMA,
            pltpu.SemaphoreType.DMA,
        ],
    )
    def _sc_rowsum(flat_hbm, out_hbm, buf0, buf1, acc_v, sem0, sem1):
        wid = lax.axis_index("s") * _NC + lax.axis_index("c")
        base = wid * (_RPT * _C)
        bufs = (buf0, buf1)
        sems = (sem0, sem1)

        def _start(j):
            pltpu.make_async_copy(
                flat_hbm.at[pl.ds(base + j * _CH, _CH)], bufs[j % 2], sems[j % 2]
            ).start()

        def _wait(j):
            pltpu.make_async_copy(
                flat_hbm.at[pl.ds(base + j * _CH, _CH)], bufs[j % 2], sems[j % 2]
            ).wait()

        _start(0)
        for r in range(_RPT):
            accs = tuple(jnp.zeros((_L,), jnp.float32) for _ in range(_UNROLL))
            for h in range(2):
                j = 2 * r + h
                if j + 1 < _NCHK:
                    _start(j + 1)
                _wait(j)
                buf = bufs[j % 2]

                def body(i, accs, buf=buf):
                    b = i * (_UNROLL * _L)
                    return tuple(
                        a + jnp.exp(buf[pl.ds(b + k * _L, _L)] * _SCALE - _SCALE)
                        for k, a in enumerate(accs)
                    )

                accs = lax.fori_loop(0, _CH // (_UNROLL * _L), body, accs)
            tot = accs[0]
            for a in accs[1:]:
                tot = tot + a
            acc_v[pl.ds(r * _L, _L)] = tot
        pltpu.sync_copy(acc_v, out_hbm.at[pl.ds(wid * _RPT * _L, _RPT * _L)])

    return _sc_rowsum


def _combine_body(g_ref, s_ref, o_ref):
    g = g_ref[...]
    s0 = jnp.sum(s_ref[...], axis=1, keepdims=True)
    c = jnp.clip(g, -1.0 + 1e-07, 1.0 - 1e-07)
    sin_t = jnp.sqrt(1.0 - c * c)
    ctm = c * _COS_M - sin_t * _SIN_M
    ctm = jnp.where(c > _TH, ctm, c - _MM)
    t = _SCALE * ctm
    s1 = s0 - jnp.exp2(g * _A2 - _A2) + jnp.exp2(t * _LOG2E - _A2)
    nll = _SCALE + jnp.log(s1) - t
    o_ref[...] = jnp.sum(nll, axis=0, keepdims=True) * (1.0 / _B)


def _tc_combine(g, s0):
    return pl.pallas_call(
        _combine_body,
        out_shape=jax.ShapeDtypeStruct((1, 1), jnp.float32),
    )(g, s0)


_RBT = 16
_NRBT = _B // _RBT


def _tc_scan_body(lbl_ref, x_ref, s_ref, g_ref):
    x = x_ref[...]
    e = jnp.exp2(x * _A2 - _A2)
    s_ref[...] = jnp.sum(e, axis=1, keepdims=True)
    col = lax.broadcasted_iota(jnp.int32, (_RBT, _C), 1)
    m = col == lbl_ref[...]
    g_ref[...] = jnp.sum(jnp.where(m, x, 0.0), axis=1, keepdims=True)


def _tc_scan(cosine, labels2d):
    return pl.pallas_call(
        _tc_scan_body,
        grid=(_NRBT,),
        in_specs=[
            pl.BlockSpec((_RBT, 1), lambda i: (i, 0)),
            pl.BlockSpec((_RBT, _C), lambda i: (i, 0)),
        ],
        out_specs=[pl.BlockSpec((_RBT, 1), lambda i: (i, 0))] * 2,
        out_shape=[jax.ShapeDtypeStruct((_B, 1), jnp.float32)] * 2,
    )(labels2d, cosine)


def kernel(cosine, labels):
    labels = labels.astype(jnp.int32)
    s0, g = _tc_scan(cosine, labels.reshape(_B, 1))
    out = _tc_combine(g, s0)
    return out[0, 0]


# speedup vs baseline: 2.1775x; 1.0381x over previous
import functools
import math

import jax
import jax.numpy as jnp
from jax import lax
from jax.experimental import pallas as pl
from jax.experimental.pallas import tpu as pltpu
from jax.experimental.pallas import tpu_sc as plsc

_SCALE = 30.0
_MARGIN = 0.5
_COS_M = math.cos(_MARGIN)
_SIN_M = math.sin(_MARGIN)
_TH = math.cos(math.pi - _MARGIN)
_MM = math.sin(math.pi - _MARGIN) * _MARGIN
_LOG2E = 1.4426950408889634
_A2 = _SCALE * _LOG2E

_B = 1024
_C = 100000

_NC = 2
_NS = 16
_NW = _NC * _NS
_BPW = _B // _NW

@functools.cache
def _sc_gather_fn():
    mesh = plsc.VectorSubcoreMesh(core_axis_name="c", subcore_axis_name="s")

    @functools.partial(
        pl.kernel,
        mesh=mesh,
        out_type=jax.ShapeDtypeStruct((_B,), jnp.float32),
        scratch_types=[
            pltpu.VMEM((_BPW,), jnp.int32),
            pltpu.VMEM((_BPW,), jnp.int32),
            pltpu.VMEM((_BPW,), jnp.float32),
            pltpu.SemaphoreType.DMA,
        ],
    )
    def _sc_gather(flat_hbm, labels_hbm, out_hbm, lbl_v, idx_v, val_v, sem):
        wid = lax.axis_index("s") * _NC + lax.axis_index("c")
        base = wid * _BPW
        pltpu.sync_copy(labels_hbm.at[pl.ds(base, _BPW)], lbl_v)
        for j in range(_BPW // 16):
            lbl = lbl_v[pl.ds(j * 16, 16)]
            rows = lax.iota(jnp.int32, 16) + (base + j * 16)
            idx_v[pl.ds(j * 16, 16)] = rows * _C + lbl
        pltpu.async_copy(flat_hbm.at[idx_v], val_v, sem).wait()
        pltpu.sync_copy(val_v, out_hbm.at[pl.ds(base, _BPW)])

    return _sc_gather


_RPT = _B // _NW
_CH = _C // 2
_NCHK = 2 * _RPT
_UNROLL = 5
_L = 16


@functools.cache
def _sc_rowsum_fn():
    mesh = plsc.VectorSubcoreMesh(core_axis_name="c", subcore_axis_name="s")

    @functools.partial(
        pl.kernel,
        mesh=mesh,
        out_type=jax.ShapeDtypeStruct((_B * _L,), jnp.float32),
        scratch_types=[
            pltpu.VMEM((_CH,), jnp.float32),
            pltpu.VMEM((_CH,), jnp.float32),
            pltpu.VMEM((_RPT * _L,), jnp.float32),
            pltpu.SemaphoreType.DMA,
            pltpu.SemaphoreType.DMA,
        ],
    )
    def _sc_rowsum(flat_hbm, out_hbm, buf0, buf1, acc_v, sem0, sem1):
        wid = lax.axis_index("s") * _NC + lax.axis_index("c")
        base = wid * (_RPT * _C)
        bufs = (buf0, buf1)
        sems = (sem0, sem1)

        def _start(j):
            pltpu.make_async_copy(
                flat_hbm.at[pl.ds(base + j * _CH, _CH)], bufs[j % 2], sems[j % 2]
            ).start()

        def _wait(j):
            pltpu.make_async_copy(
                flat_hbm.at[pl.ds(base + j * _CH, _CH)], bufs[j % 2], sems[j % 2]
            ).wait()

        _start(0)
        for r in range(_RPT):
            accs = tuple(jnp.zeros((_L,), jnp.float32) for _ in range(_UNROLL))
            for h in range(2):
                j = 2 * r + h
                if j + 1 < _NCHK:
                    _start(j + 1)
                _wait(j)
                buf = bufs[j % 2]

                def body(i, accs, buf=buf):
                    b = i * (_UNROLL * _L)
                    return tuple(
                        a + jnp.exp(buf[pl.ds(b + k * _L, _L)] * _SCALE - _SCALE)
                        for k, a in enumerate(accs)
                    )

                accs = lax.fori_loop(0, _CH // (_UNROLL * _L), body, accs)
            tot = accs[0]
            for a in accs[1:]:
                tot = tot + a
            acc_v[pl.ds(r * _L, _L)] = tot
        pltpu.sync_copy(acc_v, out_hbm.at[pl.ds(wid * _RPT * _L, _RPT * _L)])

    return _sc_rowsum


def _combine_body(g_ref, s_ref, o_ref):
    g = g_ref[...]
    s0 = jnp.sum(s_ref[...], axis=1, keepdims=True)
    c = jnp.clip(g, -1.0 + 1e-07, 1.0 - 1e-07)
    sin_t = jnp.sqrt(1.0 - c * c)
    ctm = c * _COS_M - sin_t * _SIN_M
    ctm = jnp.where(c > _TH, ctm, c - _MM)
    t = _SCALE * ctm
    s1 = s0 - jnp.exp2(g * _A2 - _A2) + jnp.exp2(t * _LOG2E - _A2)
    nll = _SCALE + jnp.log(s1) - t
    o_ref[...] = jnp.sum(nll, axis=0, keepdims=True) * (1.0 / _B)


def _tc_combine(g, s0):
    return pl.pallas_call(
        _combine_body,
        out_shape=jax.ShapeDtypeStruct((1, 1), jnp.float32),
    )(g, s0)


_RBT = 8
_TSPLIT = 4
_TBAND = _B // _TSPLIT
_NRBT = _TBAND // _RBT


def _tc_scan_body(*refs):
    lbls = refs[:_TSPLIT]
    xs = refs[_TSPLIT:2 * _TSPLIT]
    ss = refs[2 * _TSPLIT:3 * _TSPLIT]
    gs = refs[3 * _TSPLIT:]
    col = lax.broadcasted_iota(jnp.int32, (_RBT, _C), 1)
    for lbl_ref, x_ref, s_ref, g_ref in zip(lbls, xs, ss, gs):
        x = x_ref[...]
        e = jnp.exp2(x * _A2 - _A2)
        s_ref[...] = jnp.sum(e, axis=1, keepdims=True)
        m = col == lbl_ref[...]
        g_ref[...] = jnp.sum(jnp.where(m, x, 0.0), axis=1, keepdims=True)


def _tc_scan(cosine, labels2d):
    lbl_specs = [
        pl.BlockSpec((_RBT, 1), lambda i, k=k: (k * _NRBT + i, 0))
        for k in range(_TSPLIT)
    ]
    x_specs = [
        pl.BlockSpec((_RBT, _C), lambda i, k=k: (k * _NRBT + i, 0))
        for k in range(_TSPLIT)
    ]
    outs = pl.pallas_call(
        _tc_scan_body,
        grid=(_NRBT,),
        in_specs=lbl_specs + x_specs,
        out_specs=[pl.BlockSpec((_RBT, 1), lambda i: (i, 0))] * (2 * _TSPLIT),
        out_shape=[jax.ShapeDtypeStruct((_TBAND, 1), jnp.float32)] * (2 * _TSPLIT),
    )(*([labels2d] * _TSPLIT + [cosine] * _TSPLIT))
    s0 = jnp.concatenate(outs[:_TSPLIT], axis=0)
    g = jnp.concatenate(outs[_TSPLIT:], axis=0)
    return s0, g


def kernel(cosine, labels):
    labels = labels.astype(jnp.int32)
    s0, g = _tc_scan(cosine, labels.reshape(_B, 1))
    out = _tc_combine(g, s0)
    return out[0, 0]
